# baseline (speedup 1.0000x reference)
import functools

import jax
import jax.numpy as jnp
from jax import lax
from jax.experimental import pallas as pl
from jax.experimental.pallas import tpu as pltpu
from jax.experimental.pallas import tpu_sc as plsc


def _matmul_kernel(g_ref, d4_ref, o_ref):
    r = jnp.dot(g_ref[...], d4_ref[...], preferred_element_type=jnp.float32)
    o_ref[...] = r.reshape(o_ref.shape)


def _apply_d4(gp4, D4, embed):
    m4, k = gp4.shape
    e4 = D4.shape[1]
    pk = e4 // embed
    bm = 6400
    assert m4 % bm == 0
    return pl.pallas_call(
        _matmul_kernel,
        grid=(m4 // bm,),
        in_specs=[
            pl.BlockSpec((bm, k), lambda i: (i, 0)),
            pl.BlockSpec((k, e4), lambda i: (0, 0)),
        ],
        out_specs=pl.BlockSpec((bm * pk, embed), lambda i: (i, 0)),
        out_shape=jax.ShapeDtypeStruct((m4 * pk, embed), jnp.float32),
    )(gp4, D4)


def _make_gather(n, rank, n_workers, chunk, nbuf):
    per_w = n // n_workers
    n_chunks = per_w // chunk
    n_groups = n_chunks // nbuf
    assert per_w % chunk == 0 and n_chunks % nbuf == 0 and n_groups >= 2
    mesh = plsc.VectorSubcoreMesh(core_axis_name="c", subcore_axis_name="s")
    info = plsc.get_sparse_core_info()
    nc = info.num_cores

    scratch = ([pltpu.VMEM((per_w,), jnp.int32)]
               + [pltpu.VMEM((chunk, rank), jnp.float32)] * nbuf
               + [pltpu.SemaphoreType.DMA] * (2 * nbuf))

    @functools.partial(
        pl.kernel,
        mesh=mesh,
        out_type=jax.ShapeDtypeStruct((n, rank), jnp.float32),
        scratch_types=scratch,
        compiler_params=pltpu.CompilerParams(use_tc_tiling_on_sc=False),
    )
    def gather(sv_hbm, idx_hbm, out_hbm, idx_v, *bufsems):
        bufs = bufsems[:nbuf]
        gsems = bufsems[nbuf:2 * nbuf]
        wsems = bufsems[2 * nbuf:]
        wid = lax.axis_index("s") * nc + lax.axis_index("c")
        base = wid * per_w

        def fire_gather(c, b):
            pltpu.async_copy(
                sv_hbm.at[idx_v.at[pl.ds(c * chunk, chunk)]], bufs[b],
                gsems[b])

        def wait_gather(c, b):
            pltpu.make_async_copy(
                sv_hbm.at[idx_v.at[pl.ds(c * chunk, chunk)]], bufs[b],
                gsems[b]).wait()

        def fire_write(c, b):
            pltpu.async_copy(
                bufs[b], out_hbm.at[pl.ds(base + c * chunk, chunk), :],
                wsems[b])

        def wait_write(c, b):
            pltpu.make_async_copy(
                bufs[b], out_hbm.at[pl.ds(base + c * chunk, chunk), :],
                wsems[b]).wait()

        pltpu.sync_copy(idx_hbm.at[pl.ds(base, per_w)], idx_v)

        for b in range(nbuf - 1):
            fire_gather(b, b)

        wait_gather(0, 0)
        fire_write(0, 0)
        fire_gather(nbuf - 1, nbuf - 1)
        for b in range(1, nbuf):
            wait_gather(b, b)
            fire_write(b, b)
            wait_write(b - 1, (b - 1) % nbuf)
            fire_gather(b + nbuf - 1, (b - 1) % nbuf)

        def body(p, carry):
            c0 = p * nbuf
            for b in range(nbuf):
                c = c0 + b
                bp = (b + nbuf - 1) % nbuf
                wait_gather(c, b)
                fire_write(c, b)
                wait_write(c - 1, bp)
                fire_gather(c + nbuf - 1, bp)
            return carry

        lax.fori_loop(1, n_groups - 1, body, 0)

        c0 = (n_groups - 1) * nbuf
        wait_gather(c0, 0)
        fire_write(c0, 0)
        wait_write(c0 - 1, nbuf - 1)
        fire_gather(c0 + nbuf - 1, nbuf - 1)
        for b in range(1, nbuf):
            wait_gather(c0 + b, b)
            fire_write(c0 + b, b)
        for b in range(nbuf):
            wait_write(c0 + b, b)

    return gather


def kernel(SV, D, idx):
    b, l = idx.shape
    vocab, rank = SV.shape
    embed = D.shape[1]
    n = b * l
    idx_flat = idx.T.reshape(-1).astype(jnp.int32)
    gather = _make_gather(n, rank, n_workers=32, chunk=128, nbuf=5)
    gp = gather(SV, idx_flat)
    pk = 128 // rank
    gp4 = gp.reshape(n // pk, pk * rank)
    D4 = jnp.kron(jnp.eye(pk, dtype=jnp.float32), D)
    out = _apply_d4(gp4, D4, embed)
    return out.reshape(l, b, embed).transpose(1, 0, 2)


# speedup vs baseline: 1.0705x; 1.0705x over previous
import functools

import jax
import jax.numpy as jnp
from jax import lax
from jax.experimental import pallas as pl
from jax.experimental.pallas import tpu as pltpu
from jax.experimental.pallas import tpu_sc as plsc


def _linearize_kernel(svt_ref, o_ref):
    rank = svt_ref.shape[0]
    bo = o_ref.shape[0]
    pk = 128 // rank
    y = svt_ref[...].T.reshape(bo, pk, rank)
    o_ref[...] = jnp.concatenate([y[:, k, :] for k in range(pk)], axis=1)


def _linearize(SVt):
    rank, vocab = SVt.shape
    pk = 128 // rank
    bi = 12800
    bo = bi // pk
    return pl.pallas_call(
        _linearize_kernel,
        grid=(pl.cdiv(vocab, bi),),
        in_specs=[pl.BlockSpec((rank, bi), lambda i: (0, i))],
        out_specs=pl.BlockSpec((bo, 128), lambda i: (i, 0)),
        out_shape=jax.ShapeDtypeStruct((vocab // pk, 128), jnp.float32),
    )(SVt)


def _matmul_kernel(g_ref, d4_ref, o_ref):
    r = jnp.dot(g_ref[...], d4_ref[...], preferred_element_type=jnp.float32)
    o_ref[...] = r.reshape(o_ref.shape)


def _apply_d4(gp4, D4, embed):
    m4, k = gp4.shape
    e4 = D4.shape[1]
    pk = e4 // embed
    bm = 6400
    assert m4 % bm == 0
    return pl.pallas_call(
        _matmul_kernel,
        grid=(m4 // bm,),
        in_specs=[
            pl.BlockSpec((bm, k), lambda i: (i, 0)),
            pl.BlockSpec((k, e4), lambda i: (0, 0)),
        ],
        out_specs=pl.BlockSpec((bm * pk, embed), lambda i: (i, 0)),
        out_shape=jax.ShapeDtypeStruct((m4 * pk, embed), jnp.float32),
    )(gp4, D4)


def _make_gather(n, rank, n_workers, chunk, nbuf):
    per_w = n // n_workers
    n_chunks = per_w // chunk
    n_groups = n_chunks // nbuf
    assert per_w % chunk == 0 and n_chunks % nbuf == 0 and n_groups >= 2
    mesh = plsc.VectorSubcoreMesh(core_axis_name="c", subcore_axis_name="s")
    info = plsc.get_sparse_core_info()
    nc = info.num_cores

    scratch = ([pltpu.VMEM((per_w,), jnp.int32)]
               + [pltpu.VMEM((chunk, rank), jnp.float32)] * nbuf
               + [pltpu.SemaphoreType.DMA] * (2 * nbuf))

    @functools.partial(
        pl.kernel,
        mesh=mesh,
        out_type=jax.ShapeDtypeStruct((n, rank), jnp.float32),
        scratch_types=scratch,
        compiler_params=pltpu.CompilerParams(use_tc_tiling_on_sc=False),
    )
    def gather(sv_hbm, idx_hbm, out_hbm, idx_v, *bufsems):
        bufs = bufsems[:nbuf]
        gsems = bufsems[nbuf:2 * nbuf]
        wsems = bufsems[2 * nbuf:]
        wid = lax.axis_index("s") * nc + lax.axis_index("c")
        base = wid * per_w

        def fire_gather(c, b):
            pltpu.async_copy(
                sv_hbm.at[idx_v.at[pl.ds(c * chunk, chunk)]], bufs[b],
                gsems[b])

        def wait_gather(c, b):
            pltpu.make_async_copy(
                sv_hbm.at[idx_v.at[pl.ds(c * chunk, chunk)]], bufs[b],
                gsems[b]).wait()

        def fire_write(c, b):
            pltpu.async_copy(
                bufs[b], out_hbm.at[pl.ds(base + c * chunk, chunk), :],
                wsems[b])

        def wait_write(c, b):
            pltpu.make_async_copy(
                bufs[b], out_hbm.at[pl.ds(base + c * chunk, chunk), :],
                wsems[b]).wait()

        pltpu.sync_copy(idx_hbm.at[pl.ds(base, per_w)], idx_v)

        for b in range(nbuf - 1):
            fire_gather(b, b)

        wait_gather(0, 0)
        fire_write(0, 0)
        fire_gather(nbuf - 1, nbuf - 1)
        for b in range(1, nbuf):
            wait_gather(b, b)
            fire_write(b, b)
            wait_write(b - 1, (b - 1) % nbuf)
            fire_gather(b + nbuf - 1, (b - 1) % nbuf)

        def body(p, carry):
            c0 = p * nbuf
            for b in range(nbuf):
                c = c0 + b
                bp = (b + nbuf - 1) % nbuf
                wait_gather(c, b)
                fire_write(c, b)
                wait_write(c - 1, bp)
                fire_gather(c + nbuf - 1, bp)
            return carry

        lax.fori_loop(1, n_groups - 1, body, 0)

        c0 = (n_groups - 1) * nbuf
        wait_gather(c0, 0)
        fire_write(c0, 0)
        wait_write(c0 - 1, nbuf - 1)
        fire_gather(c0 + nbuf - 1, nbuf - 1)
        for b in range(1, nbuf):
            wait_gather(c0 + b, b)
            fire_write(c0 + b, b)
        for b in range(nbuf):
            wait_write(c0 + b, b)

    return gather


def kernel(SV, D, idx):
    b, l = idx.shape
    vocab, rank = SV.shape
    embed = D.shape[1]
    n = b * l
    idx_flat = idx.T.reshape(-1).astype(jnp.int32)
    gather = _make_gather(n, rank, n_workers=32, chunk=128, nbuf=5)
    sv_rows = _linearize(SV.T).reshape(-1).reshape(vocab, rank)
    gp = gather(sv_rows, idx_flat)
    pk = 128 // rank
    gp4 = gp.reshape(n // pk, pk * rank)
    D4 = jnp.kron(jnp.eye(pk, dtype=jnp.float32), D)
    out = _apply_d4(gp4, D4, embed)
    return out.reshape(l, b, embed).transpose(1, 0, 2)


# speedup vs baseline: 1.2539x; 1.1713x over previous
import functools

import jax
import jax.numpy as jnp
from jax import lax
from jax.experimental import pallas as pl
from jax.experimental.pallas import tpu as pltpu
from jax.experimental.pallas import tpu_sc as plsc


_LIN_BI = 12800


def _linearize_kernel(svt_ref, o_ref):
    rank = svt_ref.shape[0]
    bog = o_ref.shape[0]
    pk = 128 // rank
    x = svt_ref[...]
    eye = jnp.eye(rank, dtype=jnp.float32)
    acc = None
    for k in range(pk):
        ek = jnp.pad(eye, ((0, 0), (rank * k, rank * (pk - 1 - k))))
        y = lax.dot_general(x[:, k * bog:(k + 1) * bog], ek,
                            dimension_numbers=(((0,), (0,)), ((), ())),
                            preferred_element_type=jnp.float32)
        acc = y if acc is None else acc + y
    o_ref[...] = acc


def _linearize(SVt):
    rank, vocab = SVt.shape
    pk = 128 // rank
    bi = _LIN_BI
    bog = bi // pk
    n_blk = pl.cdiv(vocab, bi)
    return pl.pallas_call(
        _linearize_kernel,
        grid=(n_blk,),
        in_specs=[pl.BlockSpec((rank, bi), lambda i: (0, i))],
        out_specs=pl.BlockSpec((bog, 128), lambda i: (i, 0)),
        out_shape=jax.ShapeDtypeStruct((n_blk * bog, 128), jnp.float32),
    )(SVt)


def _matmul_kernel(g_ref, d4_ref, o_ref):
    r = jnp.dot(g_ref[...], d4_ref[...], preferred_element_type=jnp.float32)
    o_ref[...] = r.reshape(o_ref.shape)


def _apply_d4(gp4, D4, embed):
    m4, k = gp4.shape
    e4 = D4.shape[1]
    pk = e4 // embed
    bm = 6400
    assert m4 % bm == 0
    return pl.pallas_call(
        _matmul_kernel,
        grid=(m4 // bm,),
        in_specs=[
            pl.BlockSpec((bm, k), lambda i: (i, 0)),
            pl.BlockSpec((k, e4), lambda i: (0, 0)),
        ],
        out_specs=pl.BlockSpec((bm * pk, embed), lambda i: (i, 0)),
        out_shape=jax.ShapeDtypeStruct((m4 * pk, embed), jnp.float32),
    )(gp4, D4)


def _make_gather(n, rank, n_workers, chunk, nbuf):
    per_w = n // n_workers
    n_chunks = per_w // chunk
    n_groups = n_chunks // nbuf
    assert per_w % chunk == 0 and n_chunks % nbuf == 0 and n_groups >= 2
    mesh = plsc.VectorSubcoreMesh(core_axis_name="c", subcore_axis_name="s")
    info = plsc.get_sparse_core_info()
    nc = info.num_cores

    scratch = ([pltpu.VMEM((per_w,), jnp.int32)]
               + [pltpu.VMEM((chunk, rank), jnp.float32)] * nbuf
               + [pltpu.SemaphoreType.DMA] * (2 * nbuf))

    @functools.partial(
        pl.kernel,
        mesh=mesh,
        out_type=jax.ShapeDtypeStruct((n, rank), jnp.float32),
        scratch_types=scratch,
        compiler_params=pltpu.CompilerParams(use_tc_tiling_on_sc=False),
    )
    def gather(sv_hbm, idx_hbm, out_hbm, idx_v, *bufsems):
        bufs = bufsems[:nbuf]
        gsems = bufsems[nbuf:2 * nbuf]
        wsems = bufsems[2 * nbuf:]
        wid = lax.axis_index("s") * nc + lax.axis_index("c")
        base = wid * per_w

        def fire_gather(c, b):
            pltpu.async_copy(
                sv_hbm.at[idx_v.at[pl.ds(c * chunk, chunk)]], bufs[b],
                gsems[b])

        def wait_gather(c, b):
            pltpu.make_async_copy(
                sv_hbm.at[idx_v.at[pl.ds(c * chunk, chunk)]], bufs[b],
                gsems[b]).wait()

        def fire_write(c, b):
            pltpu.async_copy(
                bufs[b], out_hbm.at[pl.ds(base + c * chunk, chunk), :],
                wsems[b])

        def wait_write(c, b):
            pltpu.make_async_copy(
                bufs[b], out_hbm.at[pl.ds(base + c * chunk, chunk), :],
                wsems[b]).wait()

        pltpu.sync_copy(idx_hbm.at[pl.ds(base, per_w)], idx_v)

        for b in range(nbuf - 1):
            fire_gather(b, b)

        wait_gather(0, 0)
        fire_write(0, 0)
        fire_gather(nbuf - 1, nbuf - 1)
        for b in range(1, nbuf):
            wait_gather(b, b)
            fire_write(b, b)
            wait_write(b - 1, (b - 1) % nbuf)
            fire_gather(b + nbuf - 1, (b - 1) % nbuf)

        def body(p, carry):
            c0 = p * nbuf
            for b in range(nbuf):
                c = c0 + b
                bp = (b + nbuf - 1) % nbuf
                wait_gather(c, b)
                fire_write(c, b)
                wait_write(c - 1, bp)
                fire_gather(c + nbuf - 1, bp)
            return carry

        lax.fori_loop(1, n_groups - 1, body, 0)

        c0 = (n_groups - 1) * nbuf
        wait_gather(c0, 0)
        fire_write(c0, 0)
        wait_write(c0 - 1, nbuf - 1)
        fire_gather(c0 + nbuf - 1, nbuf - 1)
        for b in range(1, nbuf):
            wait_gather(c0 + b, b)
            fire_write(c0 + b, b)
        for b in range(nbuf):
            wait_write(c0 + b, b)

    return gather


def kernel(SV, D, idx):
    b, l = idx.shape
    vocab, rank = SV.shape
    embed = D.shape[1]
    n = b * l
    idx_flat = idx.T.reshape(-1).astype(jnp.int32)
    pk = 128 // rank
    bi = _LIN_BI
    bog = bi // pk
    blk_i = idx_flat // bi
    u = idx_flat % bi
    idx_tab = blk_i * bi + (u % bog) * pk + u // bog
    gather = _make_gather(n, rank, n_workers=32, chunk=128, nbuf=5)
    lin = _linearize(SV.T)
    vocab_pad = lin.shape[0] * pk
    sv_rows = lin.reshape(-1).reshape(vocab_pad, rank)
    gp = gather(sv_rows, idx_tab)
    gp4 = gp.reshape(n // pk, pk * rank)
    D4 = jnp.kron(jnp.eye(pk, dtype=jnp.float32), D)
    out = _apply_d4(gp4, D4, embed)
    return out.reshape(l, b, embed).transpose(1, 0, 2)


# speedup vs baseline: 1.2630x; 1.0073x over previous
import functools

import jax
import jax.numpy as jnp
from jax import lax
from jax.experimental import pallas as pl
from jax.experimental.pallas import tpu as pltpu
from jax.experimental.pallas import tpu_sc as plsc


_LIN_BI = 25600


def _linearize_kernel(svt_ref, o_ref):
    rank = svt_ref.shape[0]
    bog = o_ref.shape[0]
    pk = 128 // rank
    x = svt_ref[...]
    eye = jnp.eye(rank, dtype=jnp.float32)
    acc = None
    for k in range(pk):
        ek = jnp.pad(eye, ((0, 0), (rank * k, rank * (pk - 1 - k))))
        y = lax.dot_general(x[:, k * bog:(k + 1) * bog], ek,
                            dimension_numbers=(((0,), (0,)), ((), ())),
                            preferred_element_type=jnp.float32)
        acc = y if acc is None else acc + y
    o_ref[...] = acc


def _linearize(SVt):
    rank, vocab = SVt.shape
    pk = 128 // rank
    bi = _LIN_BI
    bog = bi // pk
    n_blk = pl.cdiv(vocab, bi)
    return pl.pallas_call(
        _linearize_kernel,
        grid=(n_blk,),
        in_specs=[pl.BlockSpec((rank, bi), lambda i: (0, i))],
        out_specs=pl.BlockSpec((bog, 128), lambda i: (i, 0)),
        out_shape=jax.ShapeDtypeStruct((n_blk * bog, 128), jnp.float32),
    )(SVt)


def _matmul_kernel(g_ref, d4_ref, o_ref):
    r = jnp.dot(g_ref[...], d4_ref[...], preferred_element_type=jnp.float32)
    o_ref[...] = r.reshape(o_ref.shape)


def _apply_d4(gp4, D4, embed):
    m4, k = gp4.shape
    e4 = D4.shape[1]
    pk = e4 // embed
    bm = 6400
    assert m4 % bm == 0
    return pl.pallas_call(
        _matmul_kernel,
        grid=(m4 // bm,),
        in_specs=[
            pl.BlockSpec((bm, k), lambda i: (i, 0)),
            pl.BlockSpec((k, e4), lambda i: (0, 0)),
        ],
        out_specs=pl.BlockSpec((bm * pk, embed), lambda i: (i, 0)),
        out_shape=jax.ShapeDtypeStruct((m4 * pk, embed), jnp.float32),
    )(gp4, D4)


def _make_gather(n, rank, n_workers, chunk, nbuf):
    per_w = n // n_workers
    n_chunks = per_w // chunk
    n_groups = n_chunks // nbuf
    assert per_w % chunk == 0 and n_chunks % nbuf == 0 and n_groups >= 2
    mesh = plsc.VectorSubcoreMesh(core_axis_name="c", subcore_axis_name="s")
    info = plsc.get_sparse_core_info()
    nc = info.num_cores

    scratch = ([pltpu.VMEM((per_w,), jnp.int32)]
               + [pltpu.VMEM((chunk, rank), jnp.float32)] * nbuf
               + [pltpu.SemaphoreType.DMA] * (2 * nbuf))

    @functools.partial(
        pl.kernel,
        mesh=mesh,
        out_type=jax.ShapeDtypeStruct((n, rank), jnp.float32),
        scratch_types=scratch,
        compiler_params=pltpu.CompilerParams(use_tc_tiling_on_sc=False),
    )
    def gather(sv_hbm, idx_hbm, out_hbm, idx_v, *bufsems):
        bufs = bufsems[:nbuf]
        gsems = bufsems[nbuf:2 * nbuf]
        wsems = bufsems[2 * nbuf:]
        wid = lax.axis_index("s") * nc + lax.axis_index("c")
        base = wid * per_w

        def fire_gather(c, b):
            pltpu.async_copy(
                sv_hbm.at[idx_v.at[pl.ds(c * chunk, chunk)]], bufs[b],
                gsems[b])

        def wait_gather(c, b):
            pltpu.make_async_copy(
                sv_hbm.at[idx_v.at[pl.ds(c * chunk, chunk)]], bufs[b],
                gsems[b]).wait()

        def fire_write(c, b):
            pltpu.async_copy(
                bufs[b], out_hbm.at[pl.ds(base + c * chunk, chunk), :],
                wsems[b])

        def wait_write(c, b):
            pltpu.make_async_copy(
                bufs[b], out_hbm.at[pl.ds(base + c * chunk, chunk), :],
                wsems[b]).wait()

        pltpu.sync_copy(idx_hbm.at[pl.ds(base, per_w)], idx_v)

        for b in range(nbuf - 1):
            fire_gather(b, b)

        wait_gather(0, 0)
        fire_write(0, 0)
        fire_gather(nbuf - 1, nbuf - 1)
        for b in range(1, nbuf):
            wait_gather(b, b)
            fire_write(b, b)
            wait_write(b - 1, (b - 1) % nbuf)
            fire_gather(b + nbuf - 1, (b - 1) % nbuf)

        def body(p, carry):
            c0 = p * nbuf
            for b in range(nbuf):
                c = c0 + b
                bp = (b + nbuf - 1) % nbuf
                wait_gather(c, b)
                fire_write(c, b)
                wait_write(c - 1, bp)
                fire_gather(c + nbuf - 1, bp)
            return carry

        lax.fori_loop(1, n_groups - 1, body, 0)

        c0 = (n_groups - 1) * nbuf
        wait_gather(c0, 0)
        fire_write(c0, 0)
        wait_write(c0 - 1, nbuf - 1)
        fire_gather(c0 + nbuf - 1, nbuf - 1)
        for b in range(1, nbuf):
            wait_gather(c0 + b, b)
            fire_write(c0 + b, b)
        for b in range(nbuf):
            wait_write(c0 + b, b)

    return gather


def kernel(SV, D, idx):
    b, l = idx.shape
    vocab, rank = SV.shape
    embed = D.shape[1]
    n = b * l
    idx_flat = idx.T.reshape(-1).astype(jnp.int32)
    pk = 128 // rank
    bi = _LIN_BI
    bog = bi // pk
    blk_i = idx_flat // bi
    u = idx_flat % bi
    idx_tab = blk_i * bi + (u % bog) * pk + u // bog
    gather = _make_gather(n, rank, n_workers=32, chunk=128, nbuf=5)
    lin = _linearize(SV.T)
    vocab_pad = lin.shape[0] * pk
    sv_rows = lin.reshape(-1).reshape(vocab_pad, rank)
    gp = gather(sv_rows, idx_tab)
    gp4 = gp.reshape(n // pk, pk * rank)
    D4 = jnp.kron(jnp.eye(pk, dtype=jnp.float32), D)
    out = _apply_d4(gp4, D4, embed)
    return out.reshape(l, b, embed).transpose(1, 0, 2)
